# baseline (speedup 1.0000x reference)
import functools

import jax
import jax.numpy as jnp
from jax import lax
from jax.experimental import pallas as pl
from jax.experimental.pallas import tpu as pltpu
from jax.experimental.pallas import tpu_sc as plsc

_ETA = 0.01
_L = 16
_NSUB = 16
_G = 16


def _dro_body(n, losses_hbm, weights_hbm, gn_hbm, out_hbm,
              loss_v, gn_v, sa_v, sb_v, part_v, w_v, out_v,
              fold_v, shared, wsem, lsem, gsem):
    sid = lax.axis_index("s")
    chunk = n // _NSUB
    base = sid * chunk

    pltpu.async_copy(losses_hbm.at[pl.ds(base, chunk)], loss_v, lsem)
    pltpu.async_copy(gn_hbm.at[pl.ds(base, chunk)], gn_v, gsem)

    zeros = jnp.zeros((_L,), jnp.float32)

    @pl.when(sid == 0)
    def _init0():
        pltpu.async_copy(weights_hbm, w_v, wsem)

    for r in range(_L):
        sa_v[pl.ds(r * _G, _G)] = zeros
        sb_v[pl.ds(r * _G, _G)] = zeros

    lane_base = jnp.arange(_L, dtype=jnp.int32) * _G
    kpack = jnp.full((_L,), 512.0, jnp.float32)

    pltpu.make_async_copy(losses_hbm.at[pl.ds(base, chunk)], loss_v,
                          lsem).wait()
    pltpu.make_async_copy(gn_hbm.at[pl.ds(base, chunk)], gn_v, gsem).wait()

    for i in range(chunk // (2 * _L)):
        off = i * 2 * _L
        lva = loss_v[pl.ds(off, _L)]
        gva = gn_v[pl.ds(off, _L)]
        lvb = loss_v[pl.ds(off + _L, _L)]
        gvb = gn_v[pl.ds(off + _L, _L)]
        plsc.addupdate_scatter(sa_v, [lane_base + gva], lva + kpack)
        plsc.addupdate_scatter(sb_v, [lane_base + gvb], lvb + kpack)

    s = jnp.zeros((_G,), jnp.float32)
    c = jnp.zeros((_G,), jnp.float32)
    for r in range(_L):
        row = sa_v[pl.ds(r * _G, _G)] + sb_v[pl.ds(r * _G, _G)]
        cr = (row * (1.0 / 512.0)).astype(jnp.int32).astype(jnp.float32)
        s = s + (row - 512.0 * cr)
        c = c + cr
    part_v[pl.ds(0, _G)] = s
    part_v[pl.ds(_G, _G)] = c

    pltpu.sync_copy(part_v, shared.at[pl.ds(sid * 2 * _G, 2 * _G)])
    plsc.subcore_barrier()

    @pl.when(sid == 0)
    def _epilogue():
        pltpu.sync_copy(shared, fold_v)
        sums = jnp.zeros((_G,), jnp.float32)
        cnts = jnp.zeros((_G,), jnp.float32)
        for t in range(_NSUB):
            sums = sums + fold_v[pl.ds(t * 2 * _G, _G)]
            cnts = cnts + fold_v[pl.ds(t * 2 * _G + _G, _G)]
        pltpu.make_async_copy(weights_hbm, w_v, wsem).wait()
        mean = jnp.where(cnts > 0.0, sums / jnp.maximum(cnts, 1.0), 0.0)
        gw = w_v[...] + _ETA * mean
        m = jnp.max(gw)
        e = jnp.exp(gw - m)
        z = jnp.sum(e)
        res = jnp.sum((e / z) * sums) * (1.0 / n)
        out_v[...] = jnp.full((_G,), res, jnp.float32)
        pltpu.sync_copy(out_v, out_hbm)


def kernel(losses, weights, group_names):
    n = losses.shape[0]
    mesh = plsc.VectorSubcoreMesh(
        core_axis_name="c", subcore_axis_name="s", num_cores=1)
    chunk = n // _NSUB
    run = pl.kernel(
        functools.partial(_dro_body, n),
        out_type=jax.ShapeDtypeStruct((_G,), jnp.float32),
        mesh=mesh,
        scratch_types=[
            pltpu.VMEM((chunk,), jnp.float32),
            pltpu.VMEM((chunk,), jnp.int32),
            pltpu.VMEM((_L * _G,), jnp.float32),
            pltpu.VMEM((_L * _G,), jnp.float32),
            pltpu.VMEM((2 * _G,), jnp.float32),
            pltpu.VMEM((_G,), jnp.float32),
            pltpu.VMEM((_G,), jnp.float32),
            pltpu.VMEM((_NSUB * 2 * _G,), jnp.float32),
            pltpu.MemorySpace.VMEM_SHARED((_NSUB * 2 * _G,), jnp.float32),
            pltpu.SemaphoreType.DMA,
            pltpu.SemaphoreType.DMA,
            pltpu.SemaphoreType.DMA,
        ],
        compiler_params=pltpu.CompilerParams(
            needs_layout_passes=False, skip_device_barrier=True),
    )
    out = run(losses, weights, group_names)
    return out[0]


# speedup vs baseline: 1.0516x; 1.0516x over previous
import functools

import jax
import jax.numpy as jnp
from jax import lax
from jax.experimental import pallas as pl
from jax.experimental.pallas import tpu as pltpu
from jax.experimental.pallas import tpu_sc as plsc

_ETA = 0.01
_L = 16
_NSUB = 16
_G = 16


def _dro_body(n, losses_hbm, weights_hbm, gn_hbm, out_hbm,
              loss_v, gn_v, sa_v, sb_v, part_v, w_v, out_v,
              fold_v, shared, wsem, lsem, gsem):
    sid = lax.axis_index("s")
    chunk = n // _NSUB
    base = sid * chunk

    pltpu.async_copy(losses_hbm.at[pl.ds(base, chunk)], loss_v, lsem)
    pltpu.async_copy(gn_hbm.at[pl.ds(base, chunk)], gn_v, gsem)

    zeros = jnp.zeros((_L,), jnp.float32)

    @pl.when(sid == 0)
    def _init0():
        pltpu.async_copy(weights_hbm, w_v, wsem)

    for r in range(_L):
        sa_v[pl.ds(r * _G, _G)] = zeros
        sb_v[pl.ds(r * _G, _G)] = zeros

    lane_base = jnp.arange(_L, dtype=jnp.int32) * _G
    kpack = jnp.full((_L,), 512.0, jnp.float32)

    pltpu.make_async_copy(losses_hbm.at[pl.ds(base, chunk)], loss_v,
                          lsem).wait()
    pltpu.make_async_copy(gn_hbm.at[pl.ds(base, chunk)], gn_v, gsem).wait()

    def load(i):
        off = pl.multiple_of(i * 2 * _L, 2 * _L)
        lva = loss_v[pl.ds(off, _L)]
        gva = gn_v[pl.ds(off, _L)]
        lvb = loss_v[pl.ds(off + _L, _L)]
        gvb = gn_v[pl.ds(off + _L, _L)]
        return lva + kpack, lane_base + gva, lvb + kpack, lane_base + gvb

    def body(i, carry):
        vala, idxa, valb, idxb = carry
        nxt = load(i + 1)
        plsc.addupdate_scatter(sa_v, [idxa], vala)
        plsc.addupdate_scatter(sb_v, [idxb], valb)
        return nxt

    niter = chunk // (2 * _L)
    vala, idxa, valb, idxb = lax.fori_loop(
        0, niter - 1, body, load(0), unroll=21)
    plsc.addupdate_scatter(sa_v, [idxa], vala)
    plsc.addupdate_scatter(sb_v, [idxb], valb)

    s = jnp.zeros((_G,), jnp.float32)
    c = jnp.zeros((_G,), jnp.float32)
    for r in range(_L):
        row = sa_v[pl.ds(r * _G, _G)] + sb_v[pl.ds(r * _G, _G)]
        cr = (row * (1.0 / 512.0)).astype(jnp.int32).astype(jnp.float32)
        s = s + (row - 512.0 * cr)
        c = c + cr
    part_v[pl.ds(0, _G)] = s
    part_v[pl.ds(_G, _G)] = c

    pltpu.sync_copy(part_v, shared.at[pl.ds(sid * 2 * _G, 2 * _G)])
    plsc.subcore_barrier()

    @pl.when(sid == 0)
    def _epilogue():
        pltpu.sync_copy(shared, fold_v)
        sums = jnp.zeros((_G,), jnp.float32)
        cnts = jnp.zeros((_G,), jnp.float32)
        for t in range(_NSUB):
            sums = sums + fold_v[pl.ds(t * 2 * _G, _G)]
            cnts = cnts + fold_v[pl.ds(t * 2 * _G + _G, _G)]
        pltpu.make_async_copy(weights_hbm, w_v, wsem).wait()
        mean = jnp.where(cnts > 0.0, sums / jnp.maximum(cnts, 1.0), 0.0)
        gw = w_v[...] + _ETA * mean
        m = jnp.max(gw)
        e = jnp.exp(gw - m)
        z = jnp.sum(e)
        res = jnp.sum((e / z) * sums) * (1.0 / n)
        out_v[...] = jnp.full((_G,), res, jnp.float32)
        pltpu.sync_copy(out_v, out_hbm)


def kernel(losses, weights, group_names):
    n = losses.shape[0]
    mesh = plsc.VectorSubcoreMesh(
        core_axis_name="c", subcore_axis_name="s", num_cores=1)
    chunk = n // _NSUB
    run = pl.kernel(
        functools.partial(_dro_body, n),
        out_type=jax.ShapeDtypeStruct((_G,), jnp.float32),
        mesh=mesh,
        scratch_types=[
            pltpu.VMEM((chunk,), jnp.float32),
            pltpu.VMEM((chunk,), jnp.int32),
            pltpu.VMEM((_L * _G,), jnp.float32),
            pltpu.VMEM((_L * _G,), jnp.float32),
            pltpu.VMEM((2 * _G,), jnp.float32),
            pltpu.VMEM((_G,), jnp.float32),
            pltpu.VMEM((_G,), jnp.float32),
            pltpu.VMEM((_NSUB * 2 * _G,), jnp.float32),
            pltpu.MemorySpace.VMEM_SHARED((_NSUB * 2 * _G,), jnp.float32),
            pltpu.SemaphoreType.DMA,
            pltpu.SemaphoreType.DMA,
            pltpu.SemaphoreType.DMA,
        ],
        compiler_params=pltpu.CompilerParams(
            needs_layout_passes=False, skip_device_barrier=True),
    )
    out = run(losses, weights, group_names)
    return out[0]


# speedup vs baseline: 1.0734x; 1.0207x over previous
import functools

import jax
import jax.numpy as jnp
from jax import lax
from jax.experimental import pallas as pl
from jax.experimental.pallas import tpu as pltpu
from jax.experimental.pallas import tpu_sc as plsc

_ETA = 0.01
_L = 16
_NSUB = 16
_G = 16


def _dro_body(n, losses_hbm, weights_hbm, gn_hbm, out_hbm,
              loss_v, gn_v, sa_v, sb_v, part_v, w_v, out_v,
              fold_v, shared, wsem, lsem, gsem):
    sid = lax.axis_index("s")
    chunk = n // _NSUB
    base = sid * chunk

    pltpu.async_copy(losses_hbm.at[pl.ds(base, chunk)], loss_v, lsem)
    pltpu.async_copy(gn_hbm.at[pl.ds(base, chunk)], gn_v, gsem)

    zeros = jnp.zeros((_L,), jnp.float32)

    @pl.when(sid == 0)
    def _init0():
        pltpu.async_copy(weights_hbm, w_v, wsem)

    for r in range(_L):
        sa_v[pl.ds(r * _G, _G)] = zeros
        sb_v[pl.ds(r * _G, _G)] = zeros

    lane_base = jnp.arange(_L, dtype=jnp.int32) * _G
    kpack = jnp.full((_L,), 512.0, jnp.float32)

    pltpu.make_async_copy(losses_hbm.at[pl.ds(base, chunk)], loss_v,
                          lsem).wait()
    pltpu.make_async_copy(gn_hbm.at[pl.ds(base, chunk)], gn_v, gsem).wait()

    def load(i):
        off = pl.multiple_of(i * 2 * _L, 2 * _L)
        lva = loss_v[pl.ds(off, _L)]
        gva = gn_v[pl.ds(off, _L)]
        lvb = loss_v[pl.ds(off + _L, _L)]
        gvb = gn_v[pl.ds(off + _L, _L)]
        return lva + kpack, lane_base + gva, lvb + kpack, lane_base + gvb

    def body(i, carry):
        vala, idxa, valb, idxb = carry
        nxt = load(i + 1)
        plsc.addupdate_scatter(sa_v, [idxa], vala)
        plsc.addupdate_scatter(sb_v, [idxb], valb)
        return nxt

    niter = chunk // (2 * _L)
    vala, idxa, valb, idxb = lax.fori_loop(
        0, niter - 1, body, load(0), unroll=8)
    plsc.addupdate_scatter(sa_v, [idxa], vala)
    plsc.addupdate_scatter(sb_v, [idxb], valb)

    s = jnp.zeros((_G,), jnp.float32)
    c = jnp.zeros((_G,), jnp.float32)
    for r in range(_L):
        row = sa_v[pl.ds(r * _G, _G)] + sb_v[pl.ds(r * _G, _G)]
        cr = (row * (1.0 / 512.0)).astype(jnp.int32).astype(jnp.float32)
        s = s + (row - 512.0 * cr)
        c = c + cr
    part_v[pl.ds(0, _G)] = s
    part_v[pl.ds(_G, _G)] = c

    pltpu.sync_copy(part_v, shared.at[pl.ds(sid * 2 * _G, 2 * _G)])
    plsc.subcore_barrier()

    @pl.when(sid == 0)
    def _epilogue():
        pltpu.sync_copy(shared, fold_v)
        sums = jnp.zeros((_G,), jnp.float32)
        cnts = jnp.zeros((_G,), jnp.float32)
        for t in range(_NSUB):
            sums = sums + fold_v[pl.ds(t * 2 * _G, _G)]
            cnts = cnts + fold_v[pl.ds(t * 2 * _G + _G, _G)]
        pltpu.make_async_copy(weights_hbm, w_v, wsem).wait()
        mean = jnp.where(cnts > 0.0, sums / jnp.maximum(cnts, 1.0), 0.0)
        gw = w_v[...] + _ETA * mean
        m = jnp.max(gw)
        e = jnp.exp(gw - m)
        z = jnp.sum(e)
        res = jnp.sum((e / z) * sums) * (1.0 / n)
        out_v[...] = jnp.full((_G,), res, jnp.float32)
        pltpu.sync_copy(out_v, out_hbm)


def kernel(losses, weights, group_names):
    n = losses.shape[0]
    mesh = plsc.VectorSubcoreMesh(
        core_axis_name="c", subcore_axis_name="s", num_cores=1)
    chunk = n // _NSUB
    run = pl.kernel(
        functools.partial(_dro_body, n),
        out_type=jax.ShapeDtypeStruct((_G,), jnp.float32),
        mesh=mesh,
        scratch_types=[
            pltpu.VMEM((chunk,), jnp.float32),
            pltpu.VMEM((chunk,), jnp.int32),
            pltpu.VMEM((_L * _G,), jnp.float32),
            pltpu.VMEM((_L * _G,), jnp.float32),
            pltpu.VMEM((2 * _G,), jnp.float32),
            pltpu.VMEM((_G,), jnp.float32),
            pltpu.VMEM((_G,), jnp.float32),
            pltpu.VMEM((_NSUB * 2 * _G,), jnp.float32),
            pltpu.MemorySpace.VMEM_SHARED((_NSUB * 2 * _G,), jnp.float32),
            pltpu.SemaphoreType.DMA,
            pltpu.SemaphoreType.DMA,
            pltpu.SemaphoreType.DMA,
        ],
        compiler_params=pltpu.CompilerParams(
            needs_layout_passes=False, skip_device_barrier=True),
    )
    out = run(losses, weights, group_names)
    return out[0]
